# Initial kernel scaffold; baseline (speedup 1.0000x reference)
import jax
import jax.numpy as jnp
from jax.experimental import pallas as pl


def kernel(travel_time_matrix, h3_indices, demand, time_windows, W, b):
    raise NotImplementedError("write your pallas kernel here")


# speedup vs baseline: 7.1692x; 7.1692x over previous
import functools
import jax
import jax.numpy as jnp
from jax import lax
from jax.experimental import pallas as pl
from jax.experimental.pallas import tpu as pltpu
from jax.experimental.pallas import tpu_sc as plsc

B, N, H, D = 1024, 1000, 31, 128
NW = 32
NB_PER_W = B // NW
CH = 200
NCHUNK = N // CH
LG = D // 16


def _table_body(ttm_ref, w_ref, b_ref, m_ref):
    m = lax.dot_general(ttm_ref[...], w_ref[...],
                        dimension_numbers=(((2,), (0,)), ((), ())),
                        preferred_element_type=jnp.float32)
    m_ref[...] = m + b_ref[...][None]


def _build_tables(ttm, w31, bias_row):
    tb = 8
    return pl.pallas_call(
        _table_body,
        grid=(B // tb,),
        in_specs=[
            pl.BlockSpec((tb, H, H), lambda i: (i, 0, 0)),
            pl.BlockSpec((H, D), lambda i: (0, 0)),
            pl.BlockSpec((1, D), lambda i: (0, 0)),
        ],
        out_specs=pl.BlockSpec((tb, H, D), lambda i: (i, 0, 0)),
        out_shape=jax.ShapeDtypeStruct((B, H, D), jnp.float32),
    )(ttm, w31, bias_row)


def _sc_body(m_hbm, idx_hbm, dem_hbm, tw0_hbm, tw1_hbm, wx_hbm, out_hbm,
             table_v, idx_v, dem_v, tw0_v, tw1_v, out_v, wx_v):
    wid = lax.axis_index("s") * 2 + lax.axis_index("c")
    pltpu.sync_copy(wx_hbm, wx_v)
    w31c = [wx_v[pl.ds(16 * l, 16)] for l in range(LG)]
    w32c = [wx_v[pl.ds(D + 16 * l, 16)] for l in range(LG)]
    w33c = [wx_v[pl.ds(2 * D + 16 * l, 16)] for l in range(LG)]
    cols = [lax.iota(jnp.int32, 16) + 16 * l for l in range(LG)]

    def batch_body(bi, _):
        b = wid * NB_PER_W + bi
        pltpu.sync_copy(m_hbm.at[b], table_v)

        def chunk_body(ci, _):
            base = b * N + ci * CH
            pltpu.sync_copy(idx_hbm.at[pl.ds(base, CH)], idx_v)
            pltpu.sync_copy(dem_hbm.at[pl.ds(base, CH)], dem_v)
            pltpu.sync_copy(tw0_hbm.at[pl.ds(base, CH)], tw0_v)
            pltpu.sync_copy(tw1_hbm.at[pl.ds(base, CH)], tw1_v)

            def node_body(n, _):
                sn = jnp.full((16,), n, dtype=jnp.int32)
                row = plsc.load_gather(idx_v, [sn])
                dem = plsc.load_gather(dem_v, [sn])
                t0 = plsc.load_gather(tw0_v, [sn])
                t1 = plsc.load_gather(tw1_v, [sn])
                nbase = sn * D
                for l in range(LG):
                    g = plsc.load_gather(table_v, [row, cols[l]])
                    r = g + dem * w31c[l] + t0 * w32c[l] + t1 * w33c[l]
                    plsc.store_scatter(out_v, [nbase + cols[l]], r)
                return 0

            lax.fori_loop(0, CH, node_body, 0)
            pltpu.sync_copy(out_v, out_hbm.at[pl.ds(base * D, CH * D)])
            return 0

        lax.fori_loop(0, NCHUNK, chunk_body, 0)
        return 0

    lax.fori_loop(0, NB_PER_W, batch_body, 0)


_sc_call = functools.partial(
    pl.kernel,
    out_type=jax.ShapeDtypeStruct((B * N * D,), jnp.float32),
    mesh=plsc.VectorSubcoreMesh(core_axis_name="c", subcore_axis_name="s"),
    compiler_params=pltpu.CompilerParams(needs_layout_passes=False),
    scratch_types=[
        pltpu.VMEM((H, D), jnp.float32),
        pltpu.VMEM((CH,), jnp.int32),
        pltpu.VMEM((CH,), jnp.float32),
        pltpu.VMEM((CH,), jnp.float32),
        pltpu.VMEM((CH,), jnp.float32),
        pltpu.VMEM((CH * D,), jnp.float32),
        pltpu.VMEM((3 * D,), jnp.float32),
    ],
)(_sc_body)


@jax.jit
def kernel(travel_time_matrix, h3_indices, demand, time_windows, W, b):
    tables = _build_tables(travel_time_matrix, W[:H], b[None])
    idx = h3_indices.astype(jnp.int32).reshape(-1)
    dem = demand.reshape(-1)
    tw0 = time_windows[..., 0].reshape(-1)
    tw1 = time_windows[..., 1].reshape(-1)
    wx = jnp.concatenate([W[H], W[H + 1], W[H + 2]])
    out = _sc_call(tables, idx, dem, tw0, tw1, wx)
    return out.reshape(B, N, D)


# speedup vs baseline: 7.9079x; 1.1030x over previous
import functools
import jax
import jax.numpy as jnp
from jax import lax
from jax.experimental import pallas as pl
from jax.experimental.pallas import tpu as pltpu
from jax.experimental.pallas import tpu_sc as plsc

B, N, H, D = 1024, 1000, 31, 128
NW = 32
NB_PER_W = B // NW
CH = 200
NCHUNK = N // CH
LG = D // 16


def _table_body(ttm_ref, w_ref, b_ref, m_ref):
    m = lax.dot_general(ttm_ref[...], w_ref[...],
                        dimension_numbers=(((2,), (0,)), ((), ())),
                        preferred_element_type=jnp.float32)
    m_ref[...] = m + b_ref[...][None]


def _build_tables(ttm, w31, bias_row):
    tb = 8
    return pl.pallas_call(
        _table_body,
        grid=(B // tb,),
        in_specs=[
            pl.BlockSpec((tb, H, H), lambda i: (i, 0, 0)),
            pl.BlockSpec((H, D), lambda i: (0, 0)),
            pl.BlockSpec((1, D), lambda i: (0, 0)),
        ],
        out_specs=pl.BlockSpec((tb, H, D), lambda i: (i, 0, 0)),
        out_shape=jax.ShapeDtypeStruct((B, H, D), jnp.float32),
    )(ttm, w31, bias_row)


def _sc_body(m_hbm, pk_hbm, wx_hbm, out_hbm,
             table_v, in0, in1, out0, out1, wx_v,
             sin0, sin1, sout0, sout1):
    wid = lax.axis_index("s") * 2 + lax.axis_index("c")
    inb = (in0, in1)
    outb = (out0, out1)
    sin = (sin0, sin1)
    sout = (sout0, sout1)

    pltpu.sync_copy(wx_hbm, wx_v)
    w31c = [wx_v[pl.ds(16 * l, 16)] for l in range(LG)]
    w32c = [wx_v[pl.ds(D + 16 * l, 16)] for l in range(LG)]
    w33c = [wx_v[pl.ds(2 * D + 16 * l, 16)] for l in range(LG)]
    cols = [lax.iota(jnp.int32, 16) + 16 * l for l in range(LG)]
    col1 = jnp.full((16,), 1, dtype=jnp.int32)
    col2 = jnp.full((16,), 2, dtype=jnp.int32)
    col3 = jnp.full((16,), 3, dtype=jnp.int32)
    zeros = jnp.zeros((16,), dtype=jnp.int32)

    b0 = wid * NB_PER_W

    def start_in(b, ci, p):
        base = b * N + ci * CH
        pltpu.async_copy(pk_hbm.at[pl.ds(base, CH)], inb[p], sin[p])

    start_in(b0, 0, 0)

    def batch_body(bi, _):
        for sub in range(2):
            b = b0 + bi * 2 + sub
            pltpu.sync_copy(m_hbm.at[b], table_v)

            for ci in range(NCHUNK):
                q = sub * NCHUNK + ci
                p = q % 2
                base = b * N + ci * CH
                if ci + 1 < NCHUNK:
                    start_in(b, ci + 1, (q + 1) % 2)
                else:
                    @pl.when(bi * 2 + sub + 1 < NB_PER_W)
                    def _():
                        start_in(b + 1, 0, (q + 1) % 2)

                pltpu.make_async_copy(
                    pk_hbm.at[pl.ds(base, CH)], inb[p], sin[p]).wait()
                if q >= 2:
                    pltpu.make_async_copy(
                        outb[p], out_hbm.at[pl.ds(base * D, CH * D)],
                        sout[p]).wait()
                else:
                    @pl.when(bi > 0)
                    def _():
                        pltpu.make_async_copy(
                            outb[p], out_hbm.at[pl.ds(base * D, CH * D)],
                            sout[p]).wait()

                def node_body(n, sn):
                    row = plsc.load_gather(
                        inb[p], [sn, zeros]).astype(jnp.int32)
                    dem = plsc.load_gather(inb[p], [sn, col1])
                    t0 = plsc.load_gather(inb[p], [sn, col2])
                    t1 = plsc.load_gather(inb[p], [sn, col3])
                    for l in range(LG):
                        g = plsc.load_gather(table_v, [row, cols[l]])
                        r = g + dem * w31c[l] + t0 * w32c[l] + t1 * w33c[l]
                        outb[p][pl.ds(n * D + 16 * l, 16)] = r
                    return sn + 1

                lax.fori_loop(0, CH, node_body, zeros, unroll=2)
                pltpu.async_copy(
                    outb[p], out_hbm.at[pl.ds(base * D, CH * D)], sout[p])
        return 0

    lax.fori_loop(0, NB_PER_W // 2, batch_body, 0)

    tail = (b0 + NB_PER_W - 1) * N
    pltpu.make_async_copy(
        outb[0], out_hbm.at[pl.ds((tail + 4 * CH) * D, CH * D)], sout[0]).wait()
    pltpu.make_async_copy(
        outb[1], out_hbm.at[pl.ds((tail + 3 * CH) * D, CH * D)], sout[1]).wait()


_sc_call = functools.partial(
    pl.kernel,
    out_type=jax.ShapeDtypeStruct((B * N * D,), jnp.float32),
    mesh=plsc.VectorSubcoreMesh(core_axis_name="c", subcore_axis_name="s"),
    compiler_params=pltpu.CompilerParams(needs_layout_passes=False),
    scratch_types=[
        pltpu.VMEM((H, D), jnp.float32),
        pltpu.VMEM((CH, 4), jnp.float32),
        pltpu.VMEM((CH, 4), jnp.float32),
        pltpu.VMEM((CH * D,), jnp.float32),
        pltpu.VMEM((CH * D,), jnp.float32),
        pltpu.VMEM((3 * D,), jnp.float32),
        pltpu.SemaphoreType.DMA,
        pltpu.SemaphoreType.DMA,
        pltpu.SemaphoreType.DMA,
        pltpu.SemaphoreType.DMA,
    ],
)(_sc_body)


@jax.jit
def kernel(travel_time_matrix, h3_indices, demand, time_windows, W, b):
    tables = _build_tables(travel_time_matrix, W[:H], b[None])
    packed = jnp.concatenate(
        [h3_indices[..., None].astype(jnp.float32),
         demand[..., None], time_windows], axis=-1).reshape(B * N, 4)
    wx = jnp.concatenate([W[H], W[H + 1], W[H + 2]])
    out = _sc_call(tables, packed, wx)
    return out.reshape(B, N, D)


# speedup vs baseline: 9.9336x; 1.2562x over previous
import functools
import jax
import jax.numpy as jnp
from jax import lax
from jax.experimental import pallas as pl
from jax.experimental.pallas import tpu as pltpu
from jax.experimental.pallas import tpu_sc as plsc

B, N, H, D = 1024, 1000, 31, 128
NW = 32
CH = 128
NCH_W = (B * N) // (NW * CH)
RING = 5
LG = D // 16


def _table_body(ttm_ref, w_ref, b_ref, m_ref):
    m = lax.dot_general(ttm_ref[...], w_ref[...],
                        dimension_numbers=(((2,), (0,)), ((), ())),
                        preferred_element_type=jnp.float32)
    m_ref[...] = m + b_ref[...][None]


def _build_tables(ttm, w31, bias_row):
    tb = 8
    return pl.pallas_call(
        _table_body,
        grid=(B // tb,),
        in_specs=[
            pl.BlockSpec((tb, H, H), lambda i: (i, 0, 0)),
            pl.BlockSpec((H, D), lambda i: (0, 0)),
            pl.BlockSpec((1, D), lambda i: (0, 0)),
        ],
        out_specs=pl.BlockSpec((tb, H, D), lambda i: (i, 0, 0)),
        out_shape=jax.ShapeDtypeStruct((B, H, D), jnp.float32),
    )(ttm, w31, bias_row)


def _sc_body(m_hbm, ix_hbm, pk_hbm, wx_hbm, out_hbm,
             ixb, pkb, rows, wx_v, six, spk, sgat, sout):
    wid = lax.axis_index("s") * 2 + lax.axis_index("c")
    q0 = wid * NCH_W

    pltpu.sync_copy(wx_hbm, wx_v)
    w31c = [wx_v[pl.ds(16 * l, 16)] for l in range(LG)]
    w32c = [wx_v[pl.ds(D + 16 * l, 16)] for l in range(LG)]
    w33c = [wx_v[pl.ds(2 * D + 16 * l, 16)] for l in range(LG)]
    one = jnp.full((16,), 1, dtype=jnp.int32)
    two = jnp.full((16,), 2, dtype=jnp.int32)
    zeros = jnp.zeros((16,), dtype=jnp.int32)

    def start_in(q, s):
        base = (q0 + q) * CH
        pltpu.async_copy(ix_hbm.at[pl.ds(base, CH)], ixb.at[s, 0], six.at[s])
        pltpu.async_copy(
            pk_hbm.at[pl.ds(base * 4, CH * 4)], pkb.at[s, 0], spk.at[s])

    def wait_in_ix(q, s):
        base = (q0 + q) * CH
        pltpu.make_async_copy(
            ix_hbm.at[pl.ds(base, CH)], ixb.at[s, 0], six.at[s]).wait()

    def wait_in_pk(q, s):
        base = (q0 + q) * CH
        pltpu.make_async_copy(
            pk_hbm.at[pl.ds(base * 4, CH * 4)], pkb.at[s, 0], spk.at[s]).wait()

    def start_gather(s):
        pltpu.async_copy(m_hbm.at[ixb.at[s, 0]], rows.at[s], sgat.at[s])

    def wait_gather(s):
        pltpu.make_async_copy(m_hbm.at[ixb.at[s, 0]], rows.at[s],
                              sgat.at[s]).wait()

    def start_out(q, s):
        base = (q0 + q) * CH
        pltpu.async_copy(rows.at[s], out_hbm.at[pl.ds(base, CH)], sout.at[s])

    def wait_out(q, s):
        base = (q0 + q) * CH
        pltpu.make_async_copy(
            rows.at[s], out_hbm.at[pl.ds(base, CH)], sout.at[s]).wait()

    for q in range(3):
        start_in(q, q)
    for q in range(2):
        wait_in_ix(q, q)
        start_gather(q)

    def outer(i, _):
        for r in range(RING):
            q = i * RING + r
            @pl.when(q + 3 < NCH_W)
            def _():
                start_in(q + 3, (r + 3) % RING)
            @pl.when(q + 2 < NCH_W)
            def _():
                @pl.when(q >= 3)
                def _():
                    wait_out(q - 3, (r + 2) % RING)
                wait_in_ix(q + 2, (r + 2) % RING)
                start_gather((r + 2) % RING)
            wait_gather(r)
            wait_in_pk(q, r)

            def node_body(n, sn4):
                dem = plsc.load_gather(pkb.at[r, 0], [sn4])
                t0 = plsc.load_gather(pkb.at[r, 0], [sn4 | one])
                t1 = plsc.load_gather(pkb.at[r, 0], [sn4 | two])
                gs = [rows.at[r][n, pl.ds(16 * l, 16)] for l in range(LG)]
                ms = [(dem * w31c[l] + t0 * w32c[l]) + t1 * w33c[l]
                      for l in range(LG)]
                for l in range(LG):
                    rows.at[r][n, pl.ds(16 * l, 16)] = gs[l] + ms[l]
                return sn4 + 4

            lax.fori_loop(0, CH, node_body, zeros, unroll=4)
            start_out(q, r)
        return 0

    lax.fori_loop(0, NCH_W // RING, outer, 0)

    for r in range(RING):
        q = NCH_W - RING + r
        wait_out(q, r)


_sc_call = functools.partial(
    pl.kernel,
    out_type=jax.ShapeDtypeStruct((B * N, D), jnp.float32),
    mesh=plsc.VectorSubcoreMesh(core_axis_name="c", subcore_axis_name="s"),
    compiler_params=pltpu.CompilerParams(needs_layout_passes=False),
    scratch_types=[
        pltpu.VMEM((RING, 1, CH), jnp.int32),
        pltpu.VMEM((RING, 1, CH * 4), jnp.float32),
        pltpu.VMEM((RING, CH, D), jnp.float32),
        pltpu.VMEM((3 * D,), jnp.float32),
        pltpu.SemaphoreType.DMA((RING,)),
        pltpu.SemaphoreType.DMA((RING,)),
        pltpu.SemaphoreType.DMA((RING,)),
        pltpu.SemaphoreType.DMA((RING,)),
    ],
)(_sc_body)


@jax.jit
def kernel(travel_time_matrix, h3_indices, demand, time_windows, W, b):
    tables = _build_tables(travel_time_matrix, W[:H], b[None])
    gidx = (h3_indices.astype(jnp.int32)
            + H * jnp.arange(B, dtype=jnp.int32)[:, None]).reshape(B * N)
    packed = jnp.concatenate(
        [demand[..., None], time_windows,
         jnp.zeros((B, N, 1), jnp.float32)], axis=-1).reshape(B * N * 4)
    wx = jnp.concatenate([W[H], W[H + 1], W[H + 2]])
    out = _sc_call(tables.reshape(B * H, D), gidx, packed, wx)
    return out.reshape(B, N, D)


# speedup vs baseline: 19.6034x; 1.9735x over previous
import functools
import jax
import jax.numpy as jnp
from jax import lax
from jax.experimental import pallas as pl
from jax.experimental.pallas import tpu as pltpu
from jax.experimental.pallas import tpu_sc as plsc

B, N, H, D = 1024, 1000, 31, 128
NW = 32
NB_PER_W = B // NW
CH = 200
NCHUNK = N // CH
LG = D // 16


def _table_body(ttm_ref, w_ref, b_ref, m_ref):
    m = lax.dot_general(ttm_ref[...], w_ref[...],
                        dimension_numbers=(((2,), (0,)), ((), ())),
                        preferred_element_type=jnp.float32)
    m_ref[...] = m + b_ref[...][None]


def _build_tables(ttm, w31, bias_row):
    tb = 8
    return pl.pallas_call(
        _table_body,
        grid=(B // tb,),
        in_specs=[
            pl.BlockSpec((tb, H, H), lambda i: (i, 0, 0)),
            pl.BlockSpec((H, D), lambda i: (0, 0)),
            pl.BlockSpec((1, D), lambda i: (0, 0)),
        ],
        out_specs=pl.BlockSpec((tb, H, D), lambda i: (i, 0, 0)),
        out_shape=jax.ShapeDtypeStruct((B, H, D), jnp.float32),
    )(ttm, w31, bias_row)


def _sc_body(m_hbm, ix_hbm, pk_hbm, wx_hbm, out_hbm,
             table_v, in0, in1, ix0, ix1, out0, out1, wx_v,
             sin0, sin1, sout0, sout1):
    wid = lax.axis_index("s") * 2 + lax.axis_index("c")
    inb = (in0, in1)
    ixb = (ix0, ix1)
    outb = (out0, out1)
    sin = (sin0, sin1)
    sout = (sout0, sout1)

    pltpu.sync_copy(wx_hbm, wx_v)
    w31c = [wx_v[pl.ds(16 * l, 16)] for l in range(LG)]
    w32c = [wx_v[pl.ds(D + 16 * l, 16)] for l in range(LG)]
    w33c = [wx_v[pl.ds(2 * D + 16 * l, 16)] for l in range(LG)]
    cols = [lax.iota(jnp.int32, 16) + 16 * l for l in range(LG)]
    col1 = jnp.full((16,), 1, dtype=jnp.int32)
    col2 = jnp.full((16,), 2, dtype=jnp.int32)
    col3 = jnp.full((16,), 3, dtype=jnp.int32)
    zeros = jnp.zeros((16,), dtype=jnp.int32)

    b0 = wid * NB_PER_W

    def start_in(b, ci, p):
        base = b * N + ci * CH
        pltpu.async_copy(pk_hbm.at[pl.ds(base, CH)], inb[p], sin[p])
        pltpu.async_copy(ix_hbm.at[pl.ds(base, CH)], ixb[p], sin[p])

    start_in(b0, 0, 0)

    def batch_body(bi, _):
        for sub in range(2):
            b = b0 + bi * 2 + sub
            pltpu.sync_copy(m_hbm.at[b], table_v)

            for ci in range(NCHUNK):
                q = sub * NCHUNK + ci
                p = q % 2
                base = b * N + ci * CH
                if ci + 1 < NCHUNK:
                    start_in(b, ci + 1, (q + 1) % 2)
                else:
                    @pl.when(bi * 2 + sub + 1 < NB_PER_W)
                    def _():
                        start_in(b + 1, 0, (q + 1) % 2)

                pltpu.make_async_copy(
                    pk_hbm.at[pl.ds(base, CH)], inb[p], sin[p]).wait()
                pltpu.make_async_copy(
                    ix_hbm.at[pl.ds(base, CH)], ixb[p], sin[p]).wait()
                if q >= 2:
                    pltpu.make_async_copy(
                        outb[p], out_hbm.at[pl.ds(base * D, CH * D)],
                        sout[p]).wait()
                else:
                    @pl.when(bi > 0)
                    def _():
                        pltpu.make_async_copy(
                            outb[p], out_hbm.at[pl.ds(base * D, CH * D)],
                            sout[p]).wait()

                def node_body(n, sn):
                    row = plsc.load_gather(ixb[p], [sn])
                    dem = plsc.load_gather(inb[p], [sn, zeros])
                    t0 = plsc.load_gather(inb[p], [sn, col1])
                    t1 = plsc.load_gather(inb[p], [sn, col2])
                    gs = [plsc.load_gather(table_v, [row | cols[l]])
                          for l in range(LG)]
                    ms = [(dem * w31c[l] + t0 * w32c[l]) + t1 * w33c[l]
                          for l in range(LG)]
                    for l in range(LG):
                        outb[p][pl.ds(n * D + 16 * l, 16)] = gs[l] + ms[l]
                    return sn + 1

                lax.fori_loop(0, CH, node_body, zeros, unroll=8)
                pltpu.async_copy(
                    outb[p], out_hbm.at[pl.ds(base * D, CH * D)], sout[p])
        return 0

    lax.fori_loop(0, NB_PER_W // 2, batch_body, 0)

    tail = (b0 + NB_PER_W - 1) * N
    pltpu.make_async_copy(
        outb[0], out_hbm.at[pl.ds((tail + 4 * CH) * D, CH * D)], sout[0]).wait()
    pltpu.make_async_copy(
        outb[1], out_hbm.at[pl.ds((tail + 3 * CH) * D, CH * D)], sout[1]).wait()


_sc_call = functools.partial(
    pl.kernel,
    out_type=jax.ShapeDtypeStruct((B * N * D,), jnp.float32),
    mesh=plsc.VectorSubcoreMesh(core_axis_name="c", subcore_axis_name="s"),
    compiler_params=pltpu.CompilerParams(needs_layout_passes=False),
    scratch_types=[
        pltpu.VMEM((H * D,), jnp.float32),
        pltpu.VMEM((CH, 4), jnp.float32),
        pltpu.VMEM((CH, 4), jnp.float32),
        pltpu.VMEM((CH,), jnp.int32),
        pltpu.VMEM((CH,), jnp.int32),
        pltpu.VMEM((CH * D,), jnp.float32),
        pltpu.VMEM((CH * D,), jnp.float32),
        pltpu.VMEM((3 * D,), jnp.float32),
        pltpu.SemaphoreType.DMA,
        pltpu.SemaphoreType.DMA,
        pltpu.SemaphoreType.DMA,
        pltpu.SemaphoreType.DMA,
    ],
)(_sc_body)


@jax.jit
def kernel(travel_time_matrix, h3_indices, demand, time_windows, W, b):
    tables = _build_tables(travel_time_matrix, W[:H], b[None])
    idxw = (h3_indices.astype(jnp.int32) << 7).reshape(B * N)
    packed = jnp.concatenate(
        [demand[..., None], time_windows,
         jnp.zeros((B, N, 1), jnp.float32)], axis=-1).reshape(B * N, 4)
    wx = jnp.concatenate([W[H], W[H + 1], W[H + 2]])
    out = _sc_call(tables.reshape(B, H * D), idxw, packed, wx)
    return out.reshape(B, N, D)
